# baseline (speedup 1.0000x reference)
import jax
import jax.numpy as jnp
from jax.experimental import pallas as pl
from jax.experimental.pallas import tpu as pltpu

_TOP_K = 2
_CAP = 192
_TB = 256
_EPB = 2


def _sigmoid(x):
    return 1.0 / (1.0 + jnp.exp(-x))


def _router_shared_body(x_ref, gw_ref, sg_ref, su_ref, sd_ref, segw_ref,
                        r_ref, w_ref, sh_ref, carry_ref):
    b = pl.program_id(0)
    x = x_ref[...]
    n_e = gw_ref.shape[0]

    logits = jax.lax.dot_general(x, gw_ref[...], (((1,), (1,)), ((), ())))
    m = jnp.max(logits, axis=1, keepdims=True)
    p = jnp.exp(logits - m)
    p = p / jnp.sum(p, axis=1, keepdims=True)

    iota_e = jax.lax.broadcasted_iota(jnp.int32, p.shape, 1)
    m1 = jnp.max(p, axis=1, keepdims=True)
    i1 = jnp.min(jnp.where(p == m1, iota_e, n_e), axis=1, keepdims=True)
    p2 = jnp.where(iota_e == i1, -1.0, p)
    m2 = jnp.max(p2, axis=1, keepdims=True)
    i2 = jnp.min(jnp.where(p2 == m2, iota_e, n_e), axis=1, keepdims=True)
    hot1 = iota_e == i1
    hot2 = iota_e == i2
    hot = hot1 | hot2
    w = jnp.where(hot1, m1, 0.0) + jnp.where(hot2, m2, 0.0)
    hotf = hot.astype(jnp.float32)

    @pl.when(b == 0)
    def _():
        carry_ref[...] = jnp.zeros_like(carry_ref)

    base = carry_ref[0:1, :]
    tb = x.shape[0]
    ri = jax.lax.broadcasted_iota(jnp.int32, (tb, tb), 0)
    ci = jax.lax.broadcasted_iota(jnp.int32, (tb, tb), 1)
    lt = (ci < ri).astype(jnp.float32)
    rank = base + jax.lax.dot_general(
        lt, hotf, (((1,), (0,)), ((), ())))
    carry_ref[0:1, :] = base + jnp.sum(hotf, axis=0, keepdims=True)

    r_ref[...] = jnp.where(hot, rank, -1.0)
    w_ref[...] = w

    xb = x.astype(jnp.bfloat16)
    g = jax.lax.dot_general(xb, sg_ref[...].astype(jnp.bfloat16),
                            (((1,), (1,)), ((), ())),
                            preferred_element_type=jnp.float32)
    u = jax.lax.dot_general(xb, su_ref[...].astype(jnp.bfloat16),
                            (((1,), (1,)), ((), ())),
                            preferred_element_type=jnp.float32)
    h = g * _sigmoid(g) * u
    y = jax.lax.dot_general(h.astype(jnp.bfloat16),
                            sd_ref[...].astype(jnp.bfloat16),
                            (((1,), (1,)), ((), ())),
                            preferred_element_type=jnp.float32)
    gate = _sigmoid(jnp.sum(x * segw_ref[...], axis=1, keepdims=True))
    sh_ref[...] = gate * y


def _expert_ffn(xe, eg, eu, ed):
    g = jax.lax.dot_general(xe, eg.astype(jnp.bfloat16),
                            (((1,), (1,)), ((), ())),
                            preferred_element_type=jnp.float32)
    u = jax.lax.dot_general(xe, eu.astype(jnp.bfloat16),
                            (((1,), (1,)), ((), ())),
                            preferred_element_type=jnp.float32)
    h = g * _sigmoid(g) * u
    y = jax.lax.dot_general(h.astype(jnp.bfloat16),
                            ed.astype(jnp.bfloat16),
                            (((1,), (1,)), ((), ())),
                            preferred_element_type=jnp.float32)
    return y.astype(jnp.bfloat16)


def _expert_body(xb_ref, r_ref, w_ref, sh_ref, eg_ref, eu_ref, ed_ref,
                 out_ref):
    s = pl.program_id(0)
    n_e = r_ref.shape[1]

    iota2 = jax.lax.broadcasted_iota(jnp.int32, (n_e, _EPB), 0)
    col2 = jax.lax.broadcasted_iota(jnp.int32, (n_e, _EPB), 1)
    onehot2 = (iota2 == s * _EPB + col2).astype(jnp.float32)
    r_cols = jax.lax.dot_general(
        r_ref[...], onehot2, (((1,), (0,)), ((), ())))
    w_cols = jax.lax.dot_general(
        w_ref[...], onehot2, (((1,), (0,)), ((), ())))

    xb = xb_ref[...]
    t = xb.shape[0]

    iota2c = jax.lax.broadcasted_iota(jnp.int32, (t, 2 * _CAP), 1)
    second = (iota2c >= _CAP).astype(jnp.int32)
    iota_c = (iota2c - second * _CAP).astype(jnp.bfloat16)
    sec_b = second.astype(jnp.bfloat16)
    fir_b = jnp.bfloat16(1) - sec_b
    r0 = r_cols[:, 0:1].astype(jnp.bfloat16)
    r1 = r_cols[:, 1:2].astype(jnp.bfloat16)
    mask = (r0 * fir_b + r1 * sec_b) == iota_c
    d2g = jnp.where(mask, jnp.bfloat16(1), jnp.bfloat16(0))
    w0 = w_cols[:, 0:1].astype(jnp.bfloat16)
    w1 = w_cols[:, 1:2].astype(jnp.bfloat16)
    d2 = jnp.where(mask, w0 * fir_b + w1 * sec_b, jnp.bfloat16(0))

    xe2 = jax.lax.dot_general(
        d2g, xb, (((0,), (0,)), ((), ())),
        preferred_element_type=jnp.float32).astype(jnp.bfloat16)

    y0 = _expert_ffn(xe2[:_CAP], eg_ref[0], eu_ref[0], ed_ref[0])
    y1 = _expert_ffn(xe2[_CAP:], eg_ref[1], eu_ref[1], ed_ref[1])
    yw2 = jnp.concatenate([y0, y1], axis=0)

    @pl.when(s == 0)
    def _():
        out_ref[...] = jnp.zeros_like(out_ref)

    out_ref[...] = out_ref[...] + jax.lax.dot_general(
        d2, yw2, (((1,), (0,)), ((), ())),
        preferred_element_type=jnp.float32)

    rows = sh_ref.shape[0]
    out_ref[pl.ds(s * rows, rows), :] = (
        out_ref[pl.ds(s * rows, rows), :] + sh_ref[...])


@jax.jit
def kernel(hidden_states, gate_weight, expert_gate_proj, expert_up_proj,
           expert_down_proj, shared_gate_proj, shared_up_proj,
           shared_down_proj, shared_expert_gate_weight):
    t, d_model = hidden_states.shape
    n_e = gate_weight.shape[0]
    s_inter = shared_gate_proj.shape[0]
    m_inter = expert_gate_proj.shape[1]
    nb = t // _TB

    r_enc, w_tok, shared = pl.pallas_call(
        _router_shared_body,
        grid=(nb,),
        in_specs=[
            pl.BlockSpec((_TB, d_model), lambda b: (b, 0)),
            pl.BlockSpec((n_e, d_model), lambda b: (0, 0)),
            pl.BlockSpec((s_inter, d_model), lambda b: (0, 0)),
            pl.BlockSpec((s_inter, d_model), lambda b: (0, 0)),
            pl.BlockSpec((d_model, s_inter), lambda b: (0, 0)),
            pl.BlockSpec((1, d_model), lambda b: (0, 0)),
        ],
        out_specs=[
            pl.BlockSpec((_TB, n_e), lambda b: (b, 0)),
            pl.BlockSpec((_TB, n_e), lambda b: (b, 0)),
            pl.BlockSpec((_TB, d_model), lambda b: (b, 0)),
        ],
        out_shape=[
            jax.ShapeDtypeStruct((t, n_e), jnp.float32),
            jax.ShapeDtypeStruct((t, n_e), jnp.float32),
            jax.ShapeDtypeStruct((t, d_model), jnp.float32),
        ],
        scratch_shapes=[pltpu.VMEM((8, n_e), jnp.float32)],
    )(hidden_states, gate_weight, shared_gate_proj, shared_up_proj,
      shared_down_proj, shared_expert_gate_weight)

    out = pl.pallas_call(
        _expert_body,
        grid=(n_e // _EPB,),
        in_specs=[
            pl.BlockSpec((t, d_model), lambda s: (0, 0)),
            pl.BlockSpec((t, n_e), lambda s: (0, 0)),
            pl.BlockSpec((t, n_e), lambda s: (0, 0)),
            pl.BlockSpec((t * _EPB // n_e, d_model), lambda s: (s, 0)),
            pl.BlockSpec((_EPB, m_inter, d_model), lambda s: (s, 0, 0)),
            pl.BlockSpec((_EPB, m_inter, d_model), lambda s: (s, 0, 0)),
            pl.BlockSpec((_EPB, d_model, m_inter), lambda s: (s, 0, 0)),
        ],
        out_specs=pl.BlockSpec((t, d_model), lambda s: (0, 0)),
        out_shape=jax.ShapeDtypeStruct((t, d_model), jnp.float32),
        compiler_params=pltpu.CompilerParams(
            vmem_limit_bytes=100 * 1024 * 1024),
    )(hidden_states.astype(jnp.bfloat16), r_enc, w_tok, shared,
      expert_gate_proj, expert_up_proj, expert_down_proj)

    return out


# speedup vs baseline: 1.0139x; 1.0139x over previous
import jax
import jax.numpy as jnp
from jax.experimental import pallas as pl
from jax.experimental.pallas import tpu as pltpu

_TOP_K = 2
_CAP = 192
_TB = 256
_EPB = 2


def _sigmoid(x):
    return 1.0 / (1.0 + jnp.exp(-x))


def _router_shared_body(x_ref, gw_ref, sg_ref, su_ref, sd_ref, segw_ref,
                        r_ref, w_ref, sh_ref, carry_ref):
    b = pl.program_id(0)
    x = x_ref[...]
    n_e = gw_ref.shape[0]

    logits = jax.lax.dot_general(x, gw_ref[...], (((1,), (1,)), ((), ())))
    m = jnp.max(logits, axis=1, keepdims=True)
    p = jnp.exp(logits - m)
    p = p / jnp.sum(p, axis=1, keepdims=True)

    iota_e = jax.lax.broadcasted_iota(jnp.int32, p.shape, 1)
    m1 = jnp.max(p, axis=1, keepdims=True)
    i1 = jnp.min(jnp.where(p == m1, iota_e, n_e), axis=1, keepdims=True)
    p2 = jnp.where(iota_e == i1, -1.0, p)
    m2 = jnp.max(p2, axis=1, keepdims=True)
    i2 = jnp.min(jnp.where(p2 == m2, iota_e, n_e), axis=1, keepdims=True)
    hot1 = iota_e == i1
    hot2 = iota_e == i2
    hot = hot1 | hot2
    w = jnp.where(hot1, m1, 0.0) + jnp.where(hot2, m2, 0.0)
    hotf = hot.astype(jnp.float32)

    @pl.when(b == 0)
    def _():
        carry_ref[...] = jnp.zeros_like(carry_ref)

    base = carry_ref[0:1, :]
    tb = x.shape[0]
    ri = jax.lax.broadcasted_iota(jnp.int32, (tb, tb), 0)
    ci = jax.lax.broadcasted_iota(jnp.int32, (tb, tb), 1)
    lt = (ci < ri).astype(jnp.float32)
    rank = base + jax.lax.dot_general(
        lt, hotf, (((1,), (0,)), ((), ())))
    carry_ref[0:1, :] = base + jnp.sum(hotf, axis=0, keepdims=True)

    r_ref[...] = jnp.where(hot, rank, -1.0)
    w_ref[...] = w

    xb = x.astype(jnp.bfloat16)
    g = jax.lax.dot_general(xb, sg_ref[...].astype(jnp.bfloat16),
                            (((1,), (1,)), ((), ())),
                            preferred_element_type=jnp.float32)
    u = jax.lax.dot_general(xb, su_ref[...].astype(jnp.bfloat16),
                            (((1,), (1,)), ((), ())),
                            preferred_element_type=jnp.float32)
    h = g * _sigmoid(g) * u
    y = jax.lax.dot_general(h.astype(jnp.bfloat16),
                            sd_ref[...].astype(jnp.bfloat16),
                            (((1,), (1,)), ((), ())),
                            preferred_element_type=jnp.float32)
    gate = _sigmoid(jnp.sum(x * segw_ref[...], axis=1, keepdims=True))
    sh_ref[...] = gate * y


def _expert_pair(xb, r_col, w_col, eg, eu, ed):
    t = r_col.shape[0]
    iota_c = jax.lax.broadcasted_iota(
        jnp.int32, (t, _CAP), 1).astype(jnp.bfloat16)
    mask = r_col.astype(jnp.bfloat16) == iota_c
    d = jnp.where(mask, jnp.bfloat16(1), jnp.bfloat16(0))
    wb = w_col.astype(jnp.bfloat16)
    dw = jnp.where(mask, wb, jnp.bfloat16(0))

    xe = jax.lax.dot_general(
        d, xb, (((0,), (0,)), ((), ())),
        preferred_element_type=jnp.float32).astype(jnp.bfloat16)
    g = jax.lax.dot_general(xe, eg.astype(jnp.bfloat16),
                            (((1,), (1,)), ((), ())),
                            preferred_element_type=jnp.float32)
    u = jax.lax.dot_general(xe, eu.astype(jnp.bfloat16),
                            (((1,), (1,)), ((), ())),
                            preferred_element_type=jnp.float32)
    h = g * _sigmoid(g) * u
    y = jax.lax.dot_general(h.astype(jnp.bfloat16),
                            ed.astype(jnp.bfloat16),
                            (((1,), (1,)), ((), ())),
                            preferred_element_type=jnp.float32)
    return dw, y.astype(jnp.bfloat16)


def _expert_body(xb_ref, r_ref, w_ref, sh_ref, eg_ref, eu_ref, ed_ref,
                 out_ref):
    s = pl.program_id(0)
    n_e = r_ref.shape[1]

    iota2 = jax.lax.broadcasted_iota(jnp.int32, (n_e, _EPB), 0)
    col2 = jax.lax.broadcasted_iota(jnp.int32, (n_e, _EPB), 1)
    onehot2 = (iota2 == s * _EPB + col2).astype(jnp.float32)
    r_cols = jax.lax.dot_general(
        r_ref[...], onehot2, (((1,), (0,)), ((), ())))
    w_cols = jax.lax.dot_general(
        w_ref[...], onehot2, (((1,), (0,)), ((), ())))

    xb = xb_ref[...]
    dw0, y0 = _expert_pair(xb, r_cols[:, 0:1], w_cols[:, 0:1],
                           eg_ref[0], eu_ref[0], ed_ref[0])
    dw1, y1 = _expert_pair(xb, r_cols[:, 1:2], w_cols[:, 1:2],
                           eg_ref[1], eu_ref[1], ed_ref[1])

    d2 = jnp.concatenate([dw0, dw1], axis=1)
    yw2 = jnp.concatenate([y0, y1], axis=0)

    @pl.when(s == 0)
    def _():
        out_ref[...] = jnp.zeros_like(out_ref)

    out_ref[...] = out_ref[...] + jax.lax.dot_general(
        d2, yw2, (((1,), (0,)), ((), ())),
        preferred_element_type=jnp.float32)

    rows = sh_ref.shape[0]
    out_ref[pl.ds(s * rows, rows), :] = (
        out_ref[pl.ds(s * rows, rows), :] + sh_ref[...])


@jax.jit
def kernel(hidden_states, gate_weight, expert_gate_proj, expert_up_proj,
           expert_down_proj, shared_gate_proj, shared_up_proj,
           shared_down_proj, shared_expert_gate_weight):
    t, d_model = hidden_states.shape
    n_e = gate_weight.shape[0]
    s_inter = shared_gate_proj.shape[0]
    m_inter = expert_gate_proj.shape[1]
    nb = t // _TB

    r_enc, w_tok, shared = pl.pallas_call(
        _router_shared_body,
        grid=(nb,),
        in_specs=[
            pl.BlockSpec((_TB, d_model), lambda b: (b, 0)),
            pl.BlockSpec((n_e, d_model), lambda b: (0, 0)),
            pl.BlockSpec((s_inter, d_model), lambda b: (0, 0)),
            pl.BlockSpec((s_inter, d_model), lambda b: (0, 0)),
            pl.BlockSpec((d_model, s_inter), lambda b: (0, 0)),
            pl.BlockSpec((1, d_model), lambda b: (0, 0)),
        ],
        out_specs=[
            pl.BlockSpec((_TB, n_e), lambda b: (b, 0)),
            pl.BlockSpec((_TB, n_e), lambda b: (b, 0)),
            pl.BlockSpec((_TB, d_model), lambda b: (b, 0)),
        ],
        out_shape=[
            jax.ShapeDtypeStruct((t, n_e), jnp.float32),
            jax.ShapeDtypeStruct((t, n_e), jnp.float32),
            jax.ShapeDtypeStruct((t, d_model), jnp.float32),
        ],
        scratch_shapes=[pltpu.VMEM((8, n_e), jnp.float32)],
    )(hidden_states, gate_weight, shared_gate_proj, shared_up_proj,
      shared_down_proj, shared_expert_gate_weight)

    out = pl.pallas_call(
        _expert_body,
        grid=(n_e // _EPB,),
        in_specs=[
            pl.BlockSpec((t, d_model), lambda s: (0, 0)),
            pl.BlockSpec((t, n_e), lambda s: (0, 0)),
            pl.BlockSpec((t, n_e), lambda s: (0, 0)),
            pl.BlockSpec((t * _EPB // n_e, d_model), lambda s: (s, 0)),
            pl.BlockSpec((_EPB, m_inter, d_model), lambda s: (s, 0, 0)),
            pl.BlockSpec((_EPB, m_inter, d_model), lambda s: (s, 0, 0)),
            pl.BlockSpec((_EPB, d_model, m_inter), lambda s: (s, 0, 0)),
        ],
        out_specs=pl.BlockSpec((t, d_model), lambda s: (0, 0)),
        out_shape=jax.ShapeDtypeStruct((t, d_model), jnp.float32),
        compiler_params=pltpu.CompilerParams(
            vmem_limit_bytes=100 * 1024 * 1024),
    )(hidden_states.astype(jnp.bfloat16), r_enc, w_tok, shared,
      expert_gate_proj, expert_up_proj, expert_down_proj)

    return out
